# baseline (speedup 1.0000x reference)
import functools

import jax
import jax.numpy as jnp
from jax import lax
from jax.experimental import pallas as pl
from jax.experimental.pallas import tpu as pltpu
from jax.experimental.pallas import tpu_sc as plsc

N_NODES = 10000
N_PAD = 10240
N_EDGES = 160000
E_PAD = 163840
D = 256
DH = 128
N_GRAPHS = 256
N_LAYERS = 3
OUT_DIM = 24
BN = 1000
GRID_N = N_NODES // BN


def _emb_body(ids_ref, emb_ref, xa_ref, xb_ref):
    ids = ids_ref[0]
    oh = (lax.broadcasted_iota(jnp.int32, (128, BN), 0)
          == jnp.broadcast_to(ids, (128, BN))).astype(jnp.float32)
    x = lax.dot_general(oh, emb_ref[...], (((0,), (0,)), ((), ())),
                        preferred_element_type=jnp.float32)
    xa_ref[...] = x[:, :DH]
    xb_ref[...] = x[:, DH:]


def _embedding(ids3, emb_pad):
    return pl.pallas_call(
        _emb_body,
        grid=(GRID_N,),
        in_specs=[
            pl.BlockSpec((1, 1, BN), lambda i: (i, 0, 0)),
            pl.BlockSpec((128, D), lambda i: (0, 0)),
        ],
        out_specs=[
            pl.BlockSpec((BN, DH), lambda i: (i, 0)),
            pl.BlockSpec((BN, DH), lambda i: (i, 0)),
        ],
        out_shape=[
            jax.ShapeDtypeStruct((N_PAD, DH), jnp.float32),
            jax.ShapeDtypeStruct((N_PAD, DH), jnp.float32),
        ],
    )(ids3, emb_pad)


def _layer_body(ha_ref, hb_ref, w1_ref, b1_ref, w2_ref, b2_ref, xa_ref, xb_ref):
    ha = ha_ref[...]
    hb = hb_ref[...]
    w1 = w1_ref[...]
    t = (jnp.dot(ha, w1[:DH], preferred_element_type=jnp.float32)
         + jnp.dot(hb, w1[DH:], preferred_element_type=jnp.float32)
         + b1_ref[...])
    t = jnp.maximum(t, 0.0)
    u = jnp.dot(t, w2_ref[...], preferred_element_type=jnp.float32) + b2_ref[...]
    u = jnp.maximum(u, 0.0)
    xa_ref[...] = u[:, :DH]
    xb_ref[...] = u[:, DH:]


def _layer_mlp(ha, hb, w1, b1, w2, b2):
    return pl.pallas_call(
        _layer_body,
        grid=(GRID_N,),
        in_specs=[
            pl.BlockSpec((BN, DH), lambda i: (i, 0)),
            pl.BlockSpec((BN, DH), lambda i: (i, 0)),
            pl.BlockSpec((D, D), lambda i: (0, 0)),
            pl.BlockSpec((1, D), lambda i: (0, 0)),
            pl.BlockSpec((D, D), lambda i: (0, 0)),
            pl.BlockSpec((1, D), lambda i: (0, 0)),
        ],
        out_specs=[
            pl.BlockSpec((BN, DH), lambda i: (i, 0)),
            pl.BlockSpec((BN, DH), lambda i: (i, 0)),
        ],
        out_shape=[
            jax.ShapeDtypeStruct((N_PAD, DH), jnp.float32),
            jax.ShapeDtypeStruct((N_PAD, DH), jnp.float32),
        ],
    )(ha, hb, w1, b1, w2, b2)


def _final_body(x1a_ref, x1b_ref, x2a_ref, x2b_ref, x3a_ref, x3b_ref,
                wjk_ref, bjk_ref, mol_ref, wout_ref, bout_ref,
                out_ref, pooled_ref):
    i = pl.program_id(0)
    wjk = wjk_ref[...]
    xc = (jnp.dot(x1a_ref[...], wjk[0 * DH:1 * DH], preferred_element_type=jnp.float32)
          + jnp.dot(x1b_ref[...], wjk[1 * DH:2 * DH], preferred_element_type=jnp.float32)
          + jnp.dot(x2a_ref[...], wjk[2 * DH:3 * DH], preferred_element_type=jnp.float32)
          + jnp.dot(x2b_ref[...], wjk[3 * DH:4 * DH], preferred_element_type=jnp.float32)
          + jnp.dot(x3a_ref[...], wjk[4 * DH:5 * DH], preferred_element_type=jnp.float32)
          + jnp.dot(x3b_ref[...], wjk[5 * DH:6 * DH], preferred_element_type=jnp.float32)
          + bjk_ref[...])
    mol = mol_ref[0]
    ohg = (lax.broadcasted_iota(jnp.int32, (N_GRAPHS, BN), 0)
           == jnp.broadcast_to(mol, (N_GRAPHS, BN))).astype(jnp.float32)
    contrib = lax.dot_general(ohg, xc, (((1,), (0,)), ((), ())),
                              preferred_element_type=jnp.float32)

    @pl.when(i == 0)
    def _init():
        pooled_ref[...] = contrib

    @pl.when(i > 0)
    def _acc():
        pooled_ref[...] += contrib

    @pl.when(i == GRID_N - 1)
    def _emit():
        out_ref[...] = (jnp.dot(pooled_ref[...], wout_ref[...],
                                preferred_element_type=jnp.float32)
                        + bout_ref[...])


def _final(x1a, x1b, x2a, x2b, x3a, x3b, wjk, bjk, mol3, wout_pad, bout_pad):
    node_spec = pl.BlockSpec((BN, DH), lambda i: (i, 0))
    return pl.pallas_call(
        _final_body,
        grid=(GRID_N,),
        in_specs=[
            node_spec, node_spec, node_spec, node_spec, node_spec, node_spec,
            pl.BlockSpec((N_LAYERS * D, D), lambda i: (0, 0)),
            pl.BlockSpec((1, D), lambda i: (0, 0)),
            pl.BlockSpec((1, 1, BN), lambda i: (i, 0, 0)),
            pl.BlockSpec((D, 128), lambda i: (0, 0)),
            pl.BlockSpec((1, 128), lambda i: (0, 0)),
        ],
        out_specs=pl.BlockSpec((N_GRAPHS, 128), lambda i: (0, 0)),
        out_shape=jax.ShapeDtypeStruct((N_GRAPHS, 128), jnp.float32),
        scratch_shapes=[pltpu.VMEM((N_GRAPHS, D), jnp.float32)],
    )(x1a, x1b, x2a, x2b, x3a, x3b, wjk, bjk, mol3, wout_pad, bout_pad)


TILES = 16
ROWS_PER_TILE = N_PAD // TILES
CHUNK = 128
CHUNKS = E_PAD // (TILES * CHUNK)


def _sc_agg_body(xa_hbm, xb_hbm, src_hbm, dst_hbm, ha_hbm, hb_hbm,
                 src_v, dst_v, rows_v, shared, sem):
    c = lax.axis_index("c")
    s = lax.axis_index("s")
    r0 = s * ROWS_PER_TILE

    def run(x_hbm, out_hbm):
        pltpu.sync_copy(x_hbm.at[pl.ds(r0, ROWS_PER_TILE)],
                        shared.at[pl.ds(r0, ROWS_PER_TILE)])
        pltpu.sync_copy(src_hbm.at[s], src_v)
        pltpu.sync_copy(dst_hbm.at[s], dst_v)
        plsc.subcore_barrier()

        def chunk(g, carry):
            pltpu.async_copy(x_hbm.at[src_v.at[g]], rows_v, sem).wait()
            pltpu.sync_copy(rows_v, shared.at[dst_v.at[g]], add=True)
            return carry

        lax.fori_loop(0, CHUNKS, chunk, 0)
        plsc.subcore_barrier()
        pltpu.sync_copy(shared.at[pl.ds(r0, ROWS_PER_TILE)],
                        out_hbm.at[pl.ds(r0, ROWS_PER_TILE)])

    @pl.when(c == 0)
    def _():
        run(xa_hbm, ha_hbm)

    @pl.when(c == 1)
    def _():
        run(xb_hbm, hb_hbm)


@functools.cache
def _sc_aggregate():
    return pl.kernel(
        _sc_agg_body,
        out_type=[jax.ShapeDtypeStruct((N_PAD, DH), jnp.float32),
                  jax.ShapeDtypeStruct((N_PAD, DH), jnp.float32)],
        mesh=plsc.VectorSubcoreMesh(core_axis_name="c", subcore_axis_name="s"),
        scratch_types=[
            pltpu.VMEM((CHUNKS, CHUNK), jnp.int32),
            pltpu.VMEM((CHUNKS, CHUNK), jnp.int32),
            pltpu.VMEM((CHUNK, DH), jnp.float32),
            pltpu.VMEM_SHARED((N_PAD, DH), jnp.float32),
            pltpu.SemaphoreType.DMA,
        ],
    )


def _aggregate(xa, xb, src3, dst3):
    return _sc_aggregate()(xa, xb, src3, dst3)


def kernel(atoms_ids, edge_index, mol_ids, emb, W1, b1, W2, b2, Wjk, bjk,
           Wout, bout):
    src = edge_index[0].astype(jnp.int32)
    dst = edge_index[1].astype(jnp.int32)
    pad = E_PAD - N_EDGES
    src3 = jnp.concatenate([src, jnp.zeros((pad,), jnp.int32)]
                           ).reshape(TILES, CHUNKS, CHUNK)
    dst3 = jnp.concatenate([dst, jnp.full((pad,), N_NODES, jnp.int32)]
                           ).reshape(TILES, CHUNKS, CHUNK)

    ids3 = atoms_ids.astype(jnp.int32).reshape(GRID_N, 1, BN)
    mol3 = mol_ids.astype(jnp.int32).reshape(GRID_N, 1, BN)
    emb_pad = jnp.zeros((128, D), jnp.float32).at[:emb.shape[0]].set(emb)
    wout_pad = jnp.zeros((D, 128), jnp.float32).at[:, :OUT_DIM].set(Wout)
    bout_pad = jnp.zeros((1, 128), jnp.float32).at[0, :OUT_DIM].set(bout)

    xa, xb = _embedding(ids3, emb_pad)
    xs = []
    for l in range(N_LAYERS):
        ha, hb = _aggregate(xa, xb, src3, dst3)
        xa, xb = _layer_mlp(ha, hb, W1[l], b1[l].reshape(1, D),
                            W2[l], b2[l].reshape(1, D))
        xs.append((xa, xb))

    logits_pad = _final(xs[0][0], xs[0][1], xs[1][0], xs[1][1],
                        xs[2][0], xs[2][1],
                        Wjk, bjk.reshape(1, D), mol3, wout_pad, bout_pad)
    return logits_pad[:, :OUT_DIM]


# speedup vs baseline: 2.0046x; 2.0046x over previous
import functools

import numpy as np

import jax
import jax.numpy as jnp
from jax import lax
from jax.experimental import pallas as pl
from jax.experimental.pallas import tpu as pltpu
from jax.experimental.pallas import tpu_sc as plsc

N_NODES = 10000
N_PAD = 10240
N_EDGES = 160000
D = 256
DH = 128
N_GRAPHS = 256
N_LAYERS = 3
OUT_DIM = 24
BN = 1000
GRID_N = N_NODES // BN


def _emb_body(ids_ref, emb_ref, xa_ref, xb_ref):
    ids = ids_ref[0]
    oh = (lax.broadcasted_iota(jnp.int32, (128, BN), 0)
          == jnp.broadcast_to(ids, (128, BN))).astype(jnp.float32)
    x = lax.dot_general(oh, emb_ref[...], (((0,), (0,)), ((), ())),
                        preferred_element_type=jnp.float32)
    xa_ref[...] = x[:, :DH]
    xb_ref[...] = x[:, DH:]


def _embedding(ids3, emb_pad):
    return pl.pallas_call(
        _emb_body,
        grid=(GRID_N,),
        in_specs=[
            pl.BlockSpec((1, 1, BN), lambda i: (i, 0, 0)),
            pl.BlockSpec((128, D), lambda i: (0, 0)),
        ],
        out_specs=[
            pl.BlockSpec((BN, DH), lambda i: (i, 0)),
            pl.BlockSpec((BN, DH), lambda i: (i, 0)),
        ],
        out_shape=[
            jax.ShapeDtypeStruct((N_PAD, DH), jnp.float32),
            jax.ShapeDtypeStruct((N_PAD, DH), jnp.float32),
        ],
    )(ids3, emb_pad)


def _layer_body(ha_ref, hb_ref, w1_ref, b1_ref, w2_ref, b2_ref, xa_ref, xb_ref):
    ha = ha_ref[...]
    hb = hb_ref[...]
    w1 = w1_ref[...]
    t = (jnp.dot(ha, w1[:DH], preferred_element_type=jnp.float32)
         + jnp.dot(hb, w1[DH:], preferred_element_type=jnp.float32)
         + b1_ref[...])
    t = jnp.maximum(t, 0.0)
    u = jnp.dot(t, w2_ref[...], preferred_element_type=jnp.float32) + b2_ref[...]
    u = jnp.maximum(u, 0.0)
    xa_ref[...] = u[:, :DH]
    xb_ref[...] = u[:, DH:]


def _layer_mlp(ha, hb, w1, b1, w2, b2):
    return pl.pallas_call(
        _layer_body,
        grid=(GRID_N,),
        in_specs=[
            pl.BlockSpec((BN, DH), lambda i: (i, 0)),
            pl.BlockSpec((BN, DH), lambda i: (i, 0)),
            pl.BlockSpec((D, D), lambda i: (0, 0)),
            pl.BlockSpec((1, D), lambda i: (0, 0)),
            pl.BlockSpec((D, D), lambda i: (0, 0)),
            pl.BlockSpec((1, D), lambda i: (0, 0)),
        ],
        out_specs=[
            pl.BlockSpec((BN, DH), lambda i: (i, 0)),
            pl.BlockSpec((BN, DH), lambda i: (i, 0)),
        ],
        out_shape=[
            jax.ShapeDtypeStruct((N_PAD, DH), jnp.float32),
            jax.ShapeDtypeStruct((N_PAD, DH), jnp.float32),
        ],
    )(ha, hb, w1, b1, w2, b2)


def _final_body(x1a_ref, x1b_ref, x2a_ref, x2b_ref, x3a_ref, x3b_ref,
                wjk_ref, bjk_ref, mol_ref, wout_ref, bout_ref,
                out_ref, pooled_ref):
    i = pl.program_id(0)
    wjk = wjk_ref[...]
    xc = (jnp.dot(x1a_ref[...], wjk[0 * DH:1 * DH], preferred_element_type=jnp.float32)
          + jnp.dot(x1b_ref[...], wjk[1 * DH:2 * DH], preferred_element_type=jnp.float32)
          + jnp.dot(x2a_ref[...], wjk[2 * DH:3 * DH], preferred_element_type=jnp.float32)
          + jnp.dot(x2b_ref[...], wjk[3 * DH:4 * DH], preferred_element_type=jnp.float32)
          + jnp.dot(x3a_ref[...], wjk[4 * DH:5 * DH], preferred_element_type=jnp.float32)
          + jnp.dot(x3b_ref[...], wjk[5 * DH:6 * DH], preferred_element_type=jnp.float32)
          + bjk_ref[...])
    mol = mol_ref[0]
    ohg = (lax.broadcasted_iota(jnp.int32, (N_GRAPHS, BN), 0)
           == jnp.broadcast_to(mol, (N_GRAPHS, BN))).astype(jnp.float32)
    contrib = lax.dot_general(ohg, xc, (((1,), (0,)), ((), ())),
                              preferred_element_type=jnp.float32)

    @pl.when(i == 0)
    def _init():
        pooled_ref[...] = contrib

    @pl.when(i > 0)
    def _acc():
        pooled_ref[...] += contrib

    @pl.when(i == GRID_N - 1)
    def _emit():
        out_ref[...] = (jnp.dot(pooled_ref[...], wout_ref[...],
                                preferred_element_type=jnp.float32)
                        + bout_ref[...])


def _final(x1a, x1b, x2a, x2b, x3a, x3b, wjk, bjk, mol3, wout_pad, bout_pad):
    node_spec = pl.BlockSpec((BN, DH), lambda i: (i, 0))
    return pl.pallas_call(
        _final_body,
        grid=(GRID_N,),
        in_specs=[
            node_spec, node_spec, node_spec, node_spec, node_spec, node_spec,
            pl.BlockSpec((N_LAYERS * D, D), lambda i: (0, 0)),
            pl.BlockSpec((1, D), lambda i: (0, 0)),
            pl.BlockSpec((1, 1, BN), lambda i: (i, 0, 0)),
            pl.BlockSpec((D, 128), lambda i: (0, 0)),
            pl.BlockSpec((1, 128), lambda i: (0, 0)),
        ],
        out_specs=pl.BlockSpec((N_GRAPHS, 128), lambda i: (0, 0)),
        out_shape=jax.ShapeDtypeStruct((N_GRAPHS, 128), jnp.float32),
        scratch_shapes=[pltpu.VMEM((N_GRAPHS, D), jnp.float32)],
    )(x1a, x1b, x2a, x2b, x3a, x3b, wjk, bjk, mol3, wout_pad, bout_pad)


TILES = 16
ROWS_PER_TILE = N_PAD // TILES
CHUNK = 128
CHUNKS_TOTAL = N_EDGES // CHUNK
CHUNKS_BASE = CHUNKS_TOTAL // TILES
CHUNKS_REM = CHUNKS_TOTAL % TILES
CHUNKS_MAX = CHUNKS_BASE + 1
CHUNK_SLOT = 80

_ROW_MAP = np.minimum(
    (np.arange(TILES * CHUNK_SLOT) // CHUNK_SLOT) * CHUNKS_BASE
    + np.minimum(np.arange(TILES * CHUNK_SLOT) // CHUNK_SLOT, CHUNKS_REM)
    + np.arange(TILES * CHUNK_SLOT) % CHUNK_SLOT,
    CHUNKS_TOTAL)


def _sc_agg_body(xa_hbm, xb_hbm, src_hbm, dst_hbm, ha_hbm, hb_hbm,
                 src_v, dst_v, rows_v, shared, sem):
    c = lax.axis_index("c")
    s = lax.axis_index("s")
    r0 = s * ROWS_PER_TILE
    start = s * CHUNK_SLOT
    nt = jnp.where(s < CHUNKS_REM, CHUNKS_MAX, CHUNKS_BASE)

    def run(x_hbm, out_hbm):
        pltpu.sync_copy(x_hbm.at[pl.ds(r0, ROWS_PER_TILE)],
                        shared.at[pl.ds(r0, ROWS_PER_TILE)])
        pltpu.sync_copy(src_hbm.at[pl.ds(start, CHUNK_SLOT)], src_v)
        pltpu.sync_copy(dst_hbm.at[pl.ds(start, CHUNK_SLOT)], dst_v)
        plsc.subcore_barrier()

        def chunk(g, carry):
            pltpu.async_copy(x_hbm.at[src_v.at[g]], rows_v, sem).wait()
            pltpu.sync_copy(rows_v, shared.at[dst_v.at[g]], add=True)
            return carry

        lax.fori_loop(0, nt, chunk, 0)
        plsc.subcore_barrier()
        pltpu.sync_copy(shared.at[pl.ds(r0, ROWS_PER_TILE)],
                        out_hbm.at[pl.ds(r0, ROWS_PER_TILE)])

    @pl.when(c == 0)
    def _():
        run(xa_hbm, ha_hbm)

    @pl.when(c == 1)
    def _():
        run(xb_hbm, hb_hbm)


@functools.cache
def _sc_aggregate():
    return pl.kernel(
        _sc_agg_body,
        out_type=[jax.ShapeDtypeStruct((N_PAD, DH), jnp.float32),
                  jax.ShapeDtypeStruct((N_PAD, DH), jnp.float32)],
        mesh=plsc.VectorSubcoreMesh(core_axis_name="c", subcore_axis_name="s"),
        scratch_types=[
            pltpu.VMEM((CHUNK_SLOT, CHUNK), jnp.int32),
            pltpu.VMEM((CHUNK_SLOT, CHUNK), jnp.int32),
            pltpu.VMEM((CHUNK, DH), jnp.float32),
            pltpu.VMEM_SHARED((N_PAD, DH), jnp.float32),
            pltpu.SemaphoreType.DMA,
        ],
    )


def _aggregate(xa, xb, src3, dst3):
    return _sc_aggregate()(xa, xb, src3, dst3)


def kernel(atoms_ids, edge_index, mol_ids, emb, W1, b1, W2, b2, Wjk, bjk,
           Wout, bout):
    src = edge_index[0].astype(jnp.int32)
    dst = edge_index[1].astype(jnp.int32)
    src3 = jnp.concatenate([src, jnp.zeros((CHUNK,), jnp.int32)]
                           ).reshape(CHUNKS_TOTAL + 1, CHUNK)[_ROW_MAP]
    dst3 = jnp.concatenate([dst, jnp.zeros((CHUNK,), jnp.int32)]
                           ).reshape(CHUNKS_TOTAL + 1, CHUNK)[_ROW_MAP]

    ids3 = atoms_ids.astype(jnp.int32).reshape(GRID_N, 1, BN)
    mol3 = mol_ids.astype(jnp.int32).reshape(GRID_N, 1, BN)
    emb_pad = jnp.zeros((128, D), jnp.float32).at[:emb.shape[0]].set(emb)
    wout_pad = jnp.zeros((D, 128), jnp.float32).at[:, :OUT_DIM].set(Wout)
    bout_pad = jnp.zeros((1, 128), jnp.float32).at[0, :OUT_DIM].set(bout)

    xa, xb = _embedding(ids3, emb_pad)
    xs = []
    for l in range(N_LAYERS):
        ha, hb = _aggregate(xa, xb, src3, dst3)
        xa, xb = _layer_mlp(ha, hb, W1[l], b1[l].reshape(1, D),
                            W2[l], b2[l].reshape(1, D))
        xs.append((xa, xb))

    logits_pad = _final(xs[0][0], xs[0][1], xs[1][0], xs[1][1],
                        xs[2][0], xs[2][1],
                        Wjk, bjk.reshape(1, D), mol3, wout_pad, bout_pad)
    return logits_pad[:, :OUT_DIM]


# speedup vs baseline: 2.8097x; 1.4016x over previous
import functools

import numpy as np

import jax
import jax.numpy as jnp
from jax import lax
from jax.experimental import pallas as pl
from jax.experimental.pallas import tpu as pltpu
from jax.experimental.pallas import tpu_sc as plsc

N_NODES = 10000
N_PAD = 10240
N_EDGES = 160000
D = 256
DH = 128
N_GRAPHS = 256
N_LAYERS = 3
OUT_DIM = 24
BN = 1000
GRID_N = N_NODES // BN


def _emb_body(ids_ref, emb_ref, xa_ref, xb_ref):
    ids = ids_ref[0]
    oh = (lax.broadcasted_iota(jnp.int32, (128, BN), 0)
          == jnp.broadcast_to(ids, (128, BN))).astype(jnp.float32)
    x = lax.dot_general(oh, emb_ref[...], (((0,), (0,)), ((), ())),
                        preferred_element_type=jnp.float32)
    xa_ref[...] = x[:, :DH]
    xb_ref[...] = x[:, DH:]


def _embedding(ids3, emb_pad):
    return pl.pallas_call(
        _emb_body,
        grid=(GRID_N,),
        in_specs=[
            pl.BlockSpec((1, 1, BN), lambda i: (i, 0, 0)),
            pl.BlockSpec((128, D), lambda i: (0, 0)),
        ],
        out_specs=[
            pl.BlockSpec((BN, DH), lambda i: (i, 0)),
            pl.BlockSpec((BN, DH), lambda i: (i, 0)),
        ],
        out_shape=[
            jax.ShapeDtypeStruct((N_PAD, DH), jnp.float32),
            jax.ShapeDtypeStruct((N_PAD, DH), jnp.float32),
        ],
    )(ids3, emb_pad)


def _layer_body(ha_ref, hb_ref, w1_ref, b1_ref, w2_ref, b2_ref, xa_ref, xb_ref):
    ha = ha_ref[...]
    hb = hb_ref[...]
    w1 = w1_ref[...]
    t = (jnp.dot(ha, w1[:DH], preferred_element_type=jnp.float32)
         + jnp.dot(hb, w1[DH:], preferred_element_type=jnp.float32)
         + b1_ref[...])
    t = jnp.maximum(t, 0.0)
    u = jnp.dot(t, w2_ref[...], preferred_element_type=jnp.float32) + b2_ref[...]
    u = jnp.maximum(u, 0.0)
    xa_ref[...] = u[:, :DH]
    xb_ref[...] = u[:, DH:]


def _layer_mlp(ha, hb, w1, b1, w2, b2):
    return pl.pallas_call(
        _layer_body,
        grid=(GRID_N,),
        in_specs=[
            pl.BlockSpec((BN, DH), lambda i: (i, 0)),
            pl.BlockSpec((BN, DH), lambda i: (i, 0)),
            pl.BlockSpec((D, D), lambda i: (0, 0)),
            pl.BlockSpec((1, D), lambda i: (0, 0)),
            pl.BlockSpec((D, D), lambda i: (0, 0)),
            pl.BlockSpec((1, D), lambda i: (0, 0)),
        ],
        out_specs=[
            pl.BlockSpec((BN, DH), lambda i: (i, 0)),
            pl.BlockSpec((BN, DH), lambda i: (i, 0)),
        ],
        out_shape=[
            jax.ShapeDtypeStruct((N_PAD, DH), jnp.float32),
            jax.ShapeDtypeStruct((N_PAD, DH), jnp.float32),
        ],
    )(ha, hb, w1, b1, w2, b2)


def _final_body(x1a_ref, x1b_ref, x2a_ref, x2b_ref, x3a_ref, x3b_ref,
                wjk_ref, bjk_ref, mol_ref, wout_ref, bout_ref,
                out_ref, pooled_ref):
    i = pl.program_id(0)
    wjk = wjk_ref[...]
    xc = (jnp.dot(x1a_ref[...], wjk[0 * DH:1 * DH], preferred_element_type=jnp.float32)
          + jnp.dot(x1b_ref[...], wjk[1 * DH:2 * DH], preferred_element_type=jnp.float32)
          + jnp.dot(x2a_ref[...], wjk[2 * DH:3 * DH], preferred_element_type=jnp.float32)
          + jnp.dot(x2b_ref[...], wjk[3 * DH:4 * DH], preferred_element_type=jnp.float32)
          + jnp.dot(x3a_ref[...], wjk[4 * DH:5 * DH], preferred_element_type=jnp.float32)
          + jnp.dot(x3b_ref[...], wjk[5 * DH:6 * DH], preferred_element_type=jnp.float32)
          + bjk_ref[...])
    mol = mol_ref[0]
    ohg = (lax.broadcasted_iota(jnp.int32, (N_GRAPHS, BN), 0)
           == jnp.broadcast_to(mol, (N_GRAPHS, BN))).astype(jnp.float32)
    contrib = lax.dot_general(ohg, xc, (((1,), (0,)), ((), ())),
                              preferred_element_type=jnp.float32)

    @pl.when(i == 0)
    def _init():
        pooled_ref[...] = contrib

    @pl.when(i > 0)
    def _acc():
        pooled_ref[...] += contrib

    @pl.when(i == GRID_N - 1)
    def _emit():
        out_ref[...] = (jnp.dot(pooled_ref[...], wout_ref[...],
                                preferred_element_type=jnp.float32)
                        + bout_ref[...])


def _final(x1a, x1b, x2a, x2b, x3a, x3b, wjk, bjk, mol3, wout_pad, bout_pad):
    node_spec = pl.BlockSpec((BN, DH), lambda i: (i, 0))
    return pl.pallas_call(
        _final_body,
        grid=(GRID_N,),
        in_specs=[
            node_spec, node_spec, node_spec, node_spec, node_spec, node_spec,
            pl.BlockSpec((N_LAYERS * D, D), lambda i: (0, 0)),
            pl.BlockSpec((1, D), lambda i: (0, 0)),
            pl.BlockSpec((1, 1, BN), lambda i: (i, 0, 0)),
            pl.BlockSpec((D, 128), lambda i: (0, 0)),
            pl.BlockSpec((1, 128), lambda i: (0, 0)),
        ],
        out_specs=pl.BlockSpec((N_GRAPHS, 128), lambda i: (0, 0)),
        out_shape=jax.ShapeDtypeStruct((N_GRAPHS, 128), jnp.float32),
        scratch_shapes=[pltpu.VMEM((N_GRAPHS, D), jnp.float32)],
    )(x1a, x1b, x2a, x2b, x3a, x3b, wjk, bjk, mol3, wout_pad, bout_pad)


TILES = 16
ROWS_PER_TILE = N_PAD // TILES
CHUNK = 128
CHUNKS_TOTAL = N_EDGES // CHUNK
CHUNKS_BASE = CHUNKS_TOTAL // TILES
CHUNKS_REM = CHUNKS_TOTAL % TILES
CHUNKS_MAX = CHUNKS_BASE + 1
CHUNK_SLOT = 80

_ROW_MAP = np.minimum(
    (np.arange(TILES * CHUNK_SLOT) // CHUNK_SLOT) * CHUNKS_BASE
    + np.minimum(np.arange(TILES * CHUNK_SLOT) // CHUNK_SLOT, CHUNKS_REM)
    + np.arange(TILES * CHUNK_SLOT) % CHUNK_SLOT,
    CHUNKS_TOTAL)


NBUF = 2


def _sc_agg_body(xa_hbm, xb_hbm, src_hbm, dst_hbm, ha_hbm, hb_hbm,
                 srcp_v, dst_v, sidx0, sidx1, rows0, rows1, sem0, sem1,
                 shared):
    c = lax.axis_index("c")
    s = lax.axis_index("s")
    r0 = s * ROWS_PER_TILE
    start = s * CHUNK_SLOT
    rows = (rows0, rows1)
    sidx = (sidx0, sidx1)
    sems = (sem0, sem1)

    def run(x_hbm, out_hbm):
        pltpu.sync_copy(x_hbm.at[pl.ds(r0, ROWS_PER_TILE)],
                        shared.at[pl.ds(r0, ROWS_PER_TILE)])
        pltpu.sync_copy(src_hbm.at[pl.ds(s * (CHUNK_SLOT // 2),
                                         CHUNK_SLOT // 2)], srcp_v)
        pltpu.sync_copy(dst_hbm.at[pl.ds(start, CHUNK_SLOT)], dst_v)
        plsc.subcore_barrier()

        def gather(g, b):
            base = (g % 2) * (CHUNK // 2)
            for j in range(CHUNK // 32):
                v = srcp_v[g // 2, pl.ds(base + 16 * j, 16)]
                sidx[b][pl.ds(32 * j, 16)] = lax.bitwise_and(v, 0xFFFF)
                sidx[b][pl.ds(32 * j + 16, 16)] = lax.shift_right_logical(v, 16)
            return pltpu.async_copy(x_hbm.at[sidx[b]], rows[b], sems[b])

        def scatter(g, b):
            pltpu.sync_copy(rows[b], shared.at[dst_v.at[g]], add=True)

        descs = {}
        for g in range(NBUF):
            descs[g] = gather(g, g % NBUF)
        for g in range(CHUNKS_BASE):
            b = g % NBUF
            descs.pop(g).wait()
            scatter(g, b)
            if g + NBUF < CHUNKS_BASE:
                descs[g + NBUF] = gather(g + NBUF, b)

        @pl.when(s < CHUNKS_REM)
        def _():
            gather(CHUNKS_BASE, 0).wait()
            scatter(CHUNKS_BASE, 0)

        plsc.subcore_barrier()
        pltpu.sync_copy(shared.at[pl.ds(r0, ROWS_PER_TILE)],
                        out_hbm.at[pl.ds(r0, ROWS_PER_TILE)])

    @pl.when(c == 0)
    def _():
        run(xa_hbm, ha_hbm)

    @pl.when(c == 1)
    def _():
        run(xb_hbm, hb_hbm)


@functools.cache
def _sc_aggregate():
    return pl.kernel(
        _sc_agg_body,
        out_type=[jax.ShapeDtypeStruct((N_PAD, DH), jnp.float32),
                  jax.ShapeDtypeStruct((N_PAD, DH), jnp.float32)],
        mesh=plsc.VectorSubcoreMesh(core_axis_name="c", subcore_axis_name="s"),
        scratch_types=[
            pltpu.VMEM((CHUNK_SLOT // 2, CHUNK), jnp.int32),
            pltpu.VMEM((CHUNK_SLOT, CHUNK), jnp.int32),
            pltpu.VMEM((CHUNK,), jnp.int32),
            pltpu.VMEM((CHUNK,), jnp.int32),
            pltpu.VMEM((CHUNK, DH), jnp.float32),
            pltpu.VMEM((CHUNK, DH), jnp.float32),
            pltpu.SemaphoreType.DMA,
            pltpu.SemaphoreType.DMA,
            pltpu.VMEM_SHARED((N_PAD, DH), jnp.float32),
        ],
    )


def _aggregate(xa, xb, src3, dst3):
    return _sc_aggregate()(xa, xb, src3, dst3)


def kernel(atoms_ids, edge_index, mol_ids, emb, W1, b1, W2, b2, Wjk, bjk,
           Wout, bout):
    src = edge_index[0].astype(jnp.int32)
    dst = edge_index[1].astype(jnp.int32)
    src3 = jnp.concatenate([src, jnp.zeros((CHUNK,), jnp.int32)]
                           ).reshape(CHUNKS_TOTAL + 1, CHUNK)[_ROW_MAP]
    dst3 = jnp.concatenate([dst, jnp.zeros((CHUNK,), jnp.int32)]
                           ).reshape(CHUNKS_TOTAL + 1, CHUNK)[_ROW_MAP]
    sp = src3.reshape(TILES * CHUNK_SLOT, CHUNK // 2, 2)
    src3 = (sp[..., 0] | (sp[..., 1] << 16)
            ).reshape(TILES * CHUNK_SLOT // 2, CHUNK)
    dst3 = (dst3.reshape(TILES * CHUNK_SLOT, CHUNK // 32, 16, 2)
            .transpose(0, 1, 3, 2).reshape(TILES * CHUNK_SLOT, CHUNK))

    ids3 = atoms_ids.astype(jnp.int32).reshape(GRID_N, 1, BN)
    mol3 = mol_ids.astype(jnp.int32).reshape(GRID_N, 1, BN)
    emb_pad = jnp.zeros((128, D), jnp.float32).at[:emb.shape[0]].set(emb)
    wout_pad = jnp.zeros((D, 128), jnp.float32).at[:, :OUT_DIM].set(Wout)
    bout_pad = jnp.zeros((1, 128), jnp.float32).at[0, :OUT_DIM].set(bout)

    xa, xb = _embedding(ids3, emb_pad)
    xs = []
    for l in range(N_LAYERS):
        ha, hb = _aggregate(xa, xb, src3, dst3)
        xa, xb = _layer_mlp(ha, hb, W1[l], b1[l].reshape(1, D),
                            W2[l], b2[l].reshape(1, D))
        xs.append((xa, xb))

    logits_pad = _final(xs[0][0], xs[0][1], xs[1][0], xs[1][1],
                        xs[2][0], xs[2][1],
                        Wjk, bjk.reshape(1, D), mol3, wout_pad, bout_pad)
    return logits_pad[:, :OUT_DIM]


# speedup vs baseline: 2.8905x; 1.0288x over previous
import functools

import numpy as np

import jax
import jax.numpy as jnp
from jax import lax
from jax.experimental import pallas as pl
from jax.experimental.pallas import tpu as pltpu
from jax.experimental.pallas import tpu_sc as plsc

N_NODES = 10000
N_PAD = 10240
N_EDGES = 160000
D = 256
DH = 128
N_GRAPHS = 256
N_LAYERS = 3
OUT_DIM = 24
BN = 1000
GRID_N = N_NODES // BN


def _emb_body(ids_ref, emb_ref, xa_ref, xb_ref):
    ids = ids_ref[0]
    oh = (lax.broadcasted_iota(jnp.int32, (128, BN), 0)
          == jnp.broadcast_to(ids, (128, BN))).astype(jnp.float32)
    x = lax.dot_general(oh, emb_ref[...], (((0,), (0,)), ((), ())),
                        preferred_element_type=jnp.float32)
    xa_ref[...] = x[:, :DH]
    xb_ref[...] = x[:, DH:]


def _embedding(ids3, emb_pad):
    return pl.pallas_call(
        _emb_body,
        grid=(GRID_N,),
        in_specs=[
            pl.BlockSpec((1, 1, BN), lambda i: (i, 0, 0)),
            pl.BlockSpec((128, D), lambda i: (0, 0)),
        ],
        out_specs=[
            pl.BlockSpec((BN, DH), lambda i: (i, 0)),
            pl.BlockSpec((BN, DH), lambda i: (i, 0)),
        ],
        out_shape=[
            jax.ShapeDtypeStruct((N_PAD, DH), jnp.float32),
            jax.ShapeDtypeStruct((N_PAD, DH), jnp.float32),
        ],
    )(ids3, emb_pad)


def _layer_body(ha_ref, hb_ref, w1_ref, b1_ref, w2_ref, b2_ref, xa_ref, xb_ref):
    ha = ha_ref[...]
    hb = hb_ref[...]
    w1 = w1_ref[...]
    t = (jnp.dot(ha, w1[:DH], preferred_element_type=jnp.float32)
         + jnp.dot(hb, w1[DH:], preferred_element_type=jnp.float32)
         + b1_ref[...])
    t = jnp.maximum(t, 0.0)
    u = jnp.dot(t, w2_ref[...], preferred_element_type=jnp.float32) + b2_ref[...]
    u = jnp.maximum(u, 0.0)
    xa_ref[...] = u[:, :DH]
    xb_ref[...] = u[:, DH:]


def _layer_mlp(ha, hb, w1, b1, w2, b2):
    return pl.pallas_call(
        _layer_body,
        grid=(GRID_N,),
        in_specs=[
            pl.BlockSpec((BN, DH), lambda i: (i, 0)),
            pl.BlockSpec((BN, DH), lambda i: (i, 0)),
            pl.BlockSpec((D, D), lambda i: (0, 0)),
            pl.BlockSpec((1, D), lambda i: (0, 0)),
            pl.BlockSpec((D, D), lambda i: (0, 0)),
            pl.BlockSpec((1, D), lambda i: (0, 0)),
        ],
        out_specs=[
            pl.BlockSpec((BN, DH), lambda i: (i, 0)),
            pl.BlockSpec((BN, DH), lambda i: (i, 0)),
        ],
        out_shape=[
            jax.ShapeDtypeStruct((N_PAD, DH), jnp.float32),
            jax.ShapeDtypeStruct((N_PAD, DH), jnp.float32),
        ],
    )(ha, hb, w1, b1, w2, b2)


def _final_body(ha_ref, hb_ref, w1_ref, b1_ref, w2_ref, b2_ref,
                x1a_ref, x1b_ref, x2a_ref, x2b_ref,
                wjk_ref, bjk_ref, mol_ref, wout_ref, bout_ref,
                out_ref, pooled_ref):
    i = pl.program_id(0)
    w1 = w1_ref[...]
    t = (jnp.dot(ha_ref[...], w1[:DH], preferred_element_type=jnp.float32)
         + jnp.dot(hb_ref[...], w1[DH:], preferred_element_type=jnp.float32)
         + b1_ref[...])
    t = jnp.maximum(t, 0.0)
    u = jnp.dot(t, w2_ref[...], preferred_element_type=jnp.float32) + b2_ref[...]
    u = jnp.maximum(u, 0.0)
    wjk = wjk_ref[...]
    xc = (jnp.dot(x1a_ref[...], wjk[0 * DH:1 * DH], preferred_element_type=jnp.float32)
          + jnp.dot(x1b_ref[...], wjk[1 * DH:2 * DH], preferred_element_type=jnp.float32)
          + jnp.dot(x2a_ref[...], wjk[2 * DH:3 * DH], preferred_element_type=jnp.float32)
          + jnp.dot(x2b_ref[...], wjk[3 * DH:4 * DH], preferred_element_type=jnp.float32)
          + jnp.dot(u[:, :DH], wjk[4 * DH:5 * DH], preferred_element_type=jnp.float32)
          + jnp.dot(u[:, DH:], wjk[5 * DH:6 * DH], preferred_element_type=jnp.float32)
          + bjk_ref[...])
    mol = mol_ref[0]
    ohg = (lax.broadcasted_iota(jnp.int32, (N_GRAPHS, BN), 0)
           == jnp.broadcast_to(mol, (N_GRAPHS, BN))).astype(jnp.float32)
    contrib = lax.dot_general(ohg, xc, (((1,), (0,)), ((), ())),
                              preferred_element_type=jnp.float32)

    @pl.when(i == 0)
    def _init():
        pooled_ref[...] = contrib

    @pl.when(i > 0)
    def _acc():
        pooled_ref[...] += contrib

    @pl.when(i == GRID_N - 1)
    def _emit():
        out_ref[...] = (jnp.dot(pooled_ref[...], wout_ref[...],
                                preferred_element_type=jnp.float32)
                        + bout_ref[...])


def _final(ha, hb, w1, b1, w2, b2, x1a, x1b, x2a, x2b,
           wjk, bjk, mol3, wout_pad, bout_pad):
    node_spec = pl.BlockSpec((BN, DH), lambda i: (i, 0))
    return pl.pallas_call(
        _final_body,
        grid=(GRID_N,),
        in_specs=[
            node_spec, node_spec,
            pl.BlockSpec((D, D), lambda i: (0, 0)),
            pl.BlockSpec((1, D), lambda i: (0, 0)),
            pl.BlockSpec((D, D), lambda i: (0, 0)),
            pl.BlockSpec((1, D), lambda i: (0, 0)),
            node_spec, node_spec, node_spec, node_spec,
            pl.BlockSpec((N_LAYERS * D, D), lambda i: (0, 0)),
            pl.BlockSpec((1, D), lambda i: (0, 0)),
            pl.BlockSpec((1, 1, BN), lambda i: (i, 0, 0)),
            pl.BlockSpec((D, 128), lambda i: (0, 0)),
            pl.BlockSpec((1, 128), lambda i: (0, 0)),
        ],
        out_specs=pl.BlockSpec((N_GRAPHS, 128), lambda i: (0, 0)),
        out_shape=jax.ShapeDtypeStruct((N_GRAPHS, 128), jnp.float32),
        scratch_shapes=[pltpu.VMEM((N_GRAPHS, D), jnp.float32)],
    )(ha, hb, w1, b1, w2, b2, x1a, x1b, x2a, x2b,
      wjk, bjk, mol3, wout_pad, bout_pad)


TILES = 16
ROWS_PER_TILE = N_PAD // TILES
CHUNK = 128
CHUNKS_TOTAL = N_EDGES // CHUNK
CHUNKS_BASE = CHUNKS_TOTAL // TILES
CHUNKS_REM = CHUNKS_TOTAL % TILES
CHUNKS_MAX = CHUNKS_BASE + 1
CHUNK_SLOT = 80

_ROW_MAP = np.minimum(
    (np.arange(TILES * CHUNK_SLOT) // CHUNK_SLOT) * CHUNKS_BASE
    + np.minimum(np.arange(TILES * CHUNK_SLOT) // CHUNK_SLOT, CHUNKS_REM)
    + np.arange(TILES * CHUNK_SLOT) % CHUNK_SLOT,
    CHUNKS_TOTAL)


NBUF = 2


def _sc_agg_body(xa_hbm, xb_hbm, src_hbm, dst_hbm, ha_hbm, hb_hbm,
                 srcp_v, dst_v, sidx0, sidx1, rows0, rows1, sem0, sem1,
                 shared):
    c = lax.axis_index("c")
    s = lax.axis_index("s")
    r0 = s * ROWS_PER_TILE
    start = s * CHUNK_SLOT
    rows = (rows0, rows1)
    sidx = (sidx0, sidx1)
    sems = (sem0, sem1)

    def run(x_hbm, out_hbm):
        pltpu.sync_copy(x_hbm.at[pl.ds(r0, ROWS_PER_TILE)],
                        shared.at[pl.ds(r0, ROWS_PER_TILE)])
        pltpu.sync_copy(src_hbm.at[pl.ds(s * (CHUNK_SLOT // 2),
                                         CHUNK_SLOT // 2)], srcp_v)
        pltpu.sync_copy(dst_hbm.at[pl.ds(start, CHUNK_SLOT)], dst_v)
        plsc.subcore_barrier()

        def gather(g, b):
            base = (g % 2) * (CHUNK // 2)
            for j in range(CHUNK // 32):
                v = srcp_v[g // 2, pl.ds(base + 16 * j, 16)]
                sidx[b][pl.ds(32 * j, 16)] = lax.bitwise_and(v, 0xFFFF)
                sidx[b][pl.ds(32 * j + 16, 16)] = lax.shift_right_logical(v, 16)
            return pltpu.async_copy(x_hbm.at[sidx[b]], rows[b], sems[b])

        def scatter(g, b):
            pltpu.sync_copy(rows[b], shared.at[dst_v.at[g]], add=True)

        descs = {}
        for g in range(NBUF):
            descs[g] = gather(g, g % NBUF)
        for g in range(CHUNKS_BASE):
            b = g % NBUF
            descs.pop(g).wait()
            scatter(g, b)
            if g + NBUF < CHUNKS_BASE:
                descs[g + NBUF] = gather(g + NBUF, b)

        @pl.when(s < CHUNKS_REM)
        def _():
            gather(CHUNKS_BASE, 0).wait()
            scatter(CHUNKS_BASE, 0)

        plsc.subcore_barrier()
        pltpu.sync_copy(shared.at[pl.ds(r0, ROWS_PER_TILE)],
                        out_hbm.at[pl.ds(r0, ROWS_PER_TILE)])

    @pl.when(c == 0)
    def _():
        run(xa_hbm, ha_hbm)

    @pl.when(c == 1)
    def _():
        run(xb_hbm, hb_hbm)


@functools.cache
def _sc_aggregate():
    return pl.kernel(
        _sc_agg_body,
        out_type=[jax.ShapeDtypeStruct((N_PAD, DH), jnp.float32),
                  jax.ShapeDtypeStruct((N_PAD, DH), jnp.float32)],
        mesh=plsc.VectorSubcoreMesh(core_axis_name="c", subcore_axis_name="s"),
        scratch_types=[
            pltpu.VMEM((CHUNK_SLOT // 2, CHUNK), jnp.int32),
            pltpu.VMEM((CHUNK_SLOT, CHUNK), jnp.int32),
            pltpu.VMEM((CHUNK,), jnp.int32),
            pltpu.VMEM((CHUNK,), jnp.int32),
            pltpu.VMEM((CHUNK, DH), jnp.float32),
            pltpu.VMEM((CHUNK, DH), jnp.float32),
            pltpu.SemaphoreType.DMA,
            pltpu.SemaphoreType.DMA,
            pltpu.VMEM_SHARED((N_PAD, DH), jnp.float32),
        ],
    )


def _aggregate(xa, xb, src3, dst3):
    return _sc_aggregate()(xa, xb, src3, dst3)


def kernel(atoms_ids, edge_index, mol_ids, emb, W1, b1, W2, b2, Wjk, bjk,
           Wout, bout):
    src = edge_index[0].astype(jnp.int32)
    dst = edge_index[1].astype(jnp.int32)
    src3 = jnp.concatenate([src, jnp.zeros((CHUNK,), jnp.int32)]
                           ).reshape(CHUNKS_TOTAL + 1, CHUNK)[_ROW_MAP]
    dst3 = jnp.concatenate([dst, jnp.zeros((CHUNK,), jnp.int32)]
                           ).reshape(CHUNKS_TOTAL + 1, CHUNK)[_ROW_MAP]
    sp = src3.reshape(TILES * CHUNK_SLOT, CHUNK // 2, 2)
    src3 = (sp[..., 0] | (sp[..., 1] << 16)
            ).reshape(TILES * CHUNK_SLOT // 2, CHUNK)
    dst3 = (dst3.reshape(TILES * CHUNK_SLOT, CHUNK // 32, 16, 2)
            .transpose(0, 1, 3, 2).reshape(TILES * CHUNK_SLOT, CHUNK))

    ids3 = atoms_ids.astype(jnp.int32).reshape(GRID_N, 1, BN)
    mol3 = mol_ids.astype(jnp.int32).reshape(GRID_N, 1, BN)
    emb_pad = jnp.zeros((128, D), jnp.float32).at[:emb.shape[0]].set(emb)
    wout_pad = jnp.zeros((D, 128), jnp.float32).at[:, :OUT_DIM].set(Wout)
    bout_pad = jnp.zeros((1, 128), jnp.float32).at[0, :OUT_DIM].set(bout)

    xa, xb = _embedding(ids3, emb_pad)
    xs = []
    for l in range(N_LAYERS - 1):
        ha, hb = _aggregate(xa, xb, src3, dst3)
        xa, xb = _layer_mlp(ha, hb, W1[l], b1[l].reshape(1, D),
                            W2[l], b2[l].reshape(1, D))
        xs.append((xa, xb))

    ha, hb = _aggregate(xa, xb, src3, dst3)
    logits_pad = _final(ha, hb, W1[2], b1[2].reshape(1, D),
                        W2[2], b2[2].reshape(1, D),
                        xs[0][0], xs[0][1], xs[1][0], xs[1][1],
                        Wjk, bjk.reshape(1, D), mol3, wout_pad, bout_pad)
    return logits_pad[:, :OUT_DIM]


# speedup vs baseline: 2.8969x; 1.0022x over previous
import functools

import numpy as np

import jax
import jax.numpy as jnp
from jax import lax
from jax.experimental import pallas as pl
from jax.experimental.pallas import tpu as pltpu
from jax.experimental.pallas import tpu_sc as plsc

N_NODES = 10000
N_PAD = 10240
N_EDGES = 160000
D = 256
DH = 128
N_GRAPHS = 256
N_LAYERS = 3
OUT_DIM = 24
BN = 1000
GRID_N = N_NODES // BN


def _emb_body(ids_ref, emb_ref, xa_ref, xb_ref):
    ids = ids_ref[0]
    oh = (lax.broadcasted_iota(jnp.int32, (128, BN), 0)
          == jnp.broadcast_to(ids, (128, BN))).astype(jnp.bfloat16)
    x = lax.dot_general(oh, emb_ref[...].astype(jnp.bfloat16),
                        (((0,), (0,)), ((), ())),
                        preferred_element_type=jnp.float32)
    xa_ref[...] = x[:, :DH]
    xb_ref[...] = x[:, DH:]


def _embedding(ids3, emb_pad):
    return pl.pallas_call(
        _emb_body,
        grid=(GRID_N,),
        in_specs=[
            pl.BlockSpec((1, 1, BN), lambda i: (i, 0, 0)),
            pl.BlockSpec((128, D), lambda i: (0, 0)),
        ],
        out_specs=[
            pl.BlockSpec((BN, DH), lambda i: (i, 0)),
            pl.BlockSpec((BN, DH), lambda i: (i, 0)),
        ],
        out_shape=[
            jax.ShapeDtypeStruct((N_PAD, DH), jnp.float32),
            jax.ShapeDtypeStruct((N_PAD, DH), jnp.float32),
        ],
    )(ids3, emb_pad)


def _layer_body(ha_ref, hb_ref, w1_ref, b1_ref, w2_ref, b2_ref, xa_ref, xb_ref):
    ha = ha_ref[...].astype(jnp.bfloat16)
    hb = hb_ref[...].astype(jnp.bfloat16)
    w1 = w1_ref[...].astype(jnp.bfloat16)
    t = (jnp.dot(ha, w1[:DH], preferred_element_type=jnp.float32)
         + jnp.dot(hb, w1[DH:], preferred_element_type=jnp.float32)
         + b1_ref[...])
    t = jnp.maximum(t, 0.0).astype(jnp.bfloat16)
    u = (jnp.dot(t, w2_ref[...].astype(jnp.bfloat16),
                 preferred_element_type=jnp.float32) + b2_ref[...])
    u = jnp.maximum(u, 0.0)
    xa_ref[...] = u[:, :DH]
    xb_ref[...] = u[:, DH:]


def _layer_mlp(ha, hb, w1, b1, w2, b2):
    return pl.pallas_call(
        _layer_body,
        grid=(GRID_N,),
        in_specs=[
            pl.BlockSpec((BN, DH), lambda i: (i, 0)),
            pl.BlockSpec((BN, DH), lambda i: (i, 0)),
            pl.BlockSpec((D, D), lambda i: (0, 0)),
            pl.BlockSpec((1, D), lambda i: (0, 0)),
            pl.BlockSpec((D, D), lambda i: (0, 0)),
            pl.BlockSpec((1, D), lambda i: (0, 0)),
        ],
        out_specs=[
            pl.BlockSpec((BN, DH), lambda i: (i, 0)),
            pl.BlockSpec((BN, DH), lambda i: (i, 0)),
        ],
        out_shape=[
            jax.ShapeDtypeStruct((N_PAD, DH), jnp.float32),
            jax.ShapeDtypeStruct((N_PAD, DH), jnp.float32),
        ],
    )(ha, hb, w1, b1, w2, b2)


def _final_body(ha_ref, hb_ref, w1_ref, b1_ref, w2_ref, b2_ref,
                x1a_ref, x1b_ref, x2a_ref, x2b_ref,
                wjk_ref, bjk_ref, mol_ref, wout_ref, bout_ref,
                out_ref, pooled_ref):
    i = pl.program_id(0)
    w1 = w1_ref[...].astype(jnp.bfloat16)
    t = (jnp.dot(ha_ref[...].astype(jnp.bfloat16), w1[:DH],
                 preferred_element_type=jnp.float32)
         + jnp.dot(hb_ref[...].astype(jnp.bfloat16), w1[DH:],
                   preferred_element_type=jnp.float32)
         + b1_ref[...])
    t = jnp.maximum(t, 0.0).astype(jnp.bfloat16)
    u = (jnp.dot(t, w2_ref[...].astype(jnp.bfloat16),
                 preferred_element_type=jnp.float32) + b2_ref[...])
    u = jnp.maximum(u, 0.0).astype(jnp.bfloat16)
    wjk = wjk_ref[...].astype(jnp.bfloat16)
    xc = (jnp.dot(x1a_ref[...].astype(jnp.bfloat16), wjk[0 * DH:1 * DH], preferred_element_type=jnp.float32)
          + jnp.dot(x1b_ref[...].astype(jnp.bfloat16), wjk[1 * DH:2 * DH], preferred_element_type=jnp.float32)
          + jnp.dot(x2a_ref[...].astype(jnp.bfloat16), wjk[2 * DH:3 * DH], preferred_element_type=jnp.float32)
          + jnp.dot(x2b_ref[...].astype(jnp.bfloat16), wjk[3 * DH:4 * DH], preferred_element_type=jnp.float32)
          + jnp.dot(u[:, :DH], wjk[4 * DH:5 * DH], preferred_element_type=jnp.float32)
          + jnp.dot(u[:, DH:], wjk[5 * DH:6 * DH], preferred_element_type=jnp.float32)
          + bjk_ref[...])
    mol = mol_ref[0]
    ohg = (lax.broadcasted_iota(jnp.int32, (N_GRAPHS, BN), 0)
           == jnp.broadcast_to(mol, (N_GRAPHS, BN))).astype(jnp.bfloat16)
    contrib = lax.dot_general(ohg, xc.astype(jnp.bfloat16),
                              (((1,), (0,)), ((), ())),
                              preferred_element_type=jnp.float32)

    @pl.when(i == 0)
    def _init():
        pooled_ref[...] = contrib

    @pl.when(i > 0)
    def _acc():
        pooled_ref[...] += contrib

    @pl.when(i == GRID_N - 1)
    def _emit():
        out_ref[...] = (jnp.dot(pooled_ref[...], wout_ref[...],
                                preferred_element_type=jnp.float32)
                        + bout_ref[...])


def _final(ha, hb, w1, b1, w2, b2, x1a, x1b, x2a, x2b,
           wjk, bjk, mol3, wout_pad, bout_pad):
    node_spec = pl.BlockSpec((BN, DH), lambda i: (i, 0))
    return pl.pallas_call(
        _final_body,
        grid=(GRID_N,),
        in_specs=[
            node_spec, node_spec,
            pl.BlockSpec((D, D), lambda i: (0, 0)),
            pl.BlockSpec((1, D), lambda i: (0, 0)),
            pl.BlockSpec((D, D), lambda i: (0, 0)),
            pl.BlockSpec((1, D), lambda i: (0, 0)),
            node_spec, node_spec, node_spec, node_spec,
            pl.BlockSpec((N_LAYERS * D, D), lambda i: (0, 0)),
            pl.BlockSpec((1, D), lambda i: (0, 0)),
            pl.BlockSpec((1, 1, BN), lambda i: (i, 0, 0)),
            pl.BlockSpec((D, 128), lambda i: (0, 0)),
            pl.BlockSpec((1, 128), lambda i: (0, 0)),
        ],
        out_specs=pl.BlockSpec((N_GRAPHS, 128), lambda i: (0, 0)),
        out_shape=jax.ShapeDtypeStruct((N_GRAPHS, 128), jnp.float32),
        scratch_shapes=[pltpu.VMEM((N_GRAPHS, D), jnp.float32)],
    )(ha, hb, w1, b1, w2, b2, x1a, x1b, x2a, x2b,
      wjk, bjk, mol3, wout_pad, bout_pad)


TILES = 16
ROWS_PER_TILE = N_PAD // TILES
CHUNK = 128
CHUNKS_TOTAL = N_EDGES // CHUNK
CHUNKS_BASE = CHUNKS_TOTAL // TILES
CHUNKS_REM = CHUNKS_TOTAL % TILES
CHUNKS_MAX = CHUNKS_BASE + 1
CHUNK_SLOT = 80

_ROW_MAP = np.minimum(
    (np.arange(TILES * CHUNK_SLOT) // CHUNK_SLOT) * CHUNKS_BASE
    + np.minimum(np.arange(TILES * CHUNK_SLOT) // CHUNK_SLOT, CHUNKS_REM)
    + np.arange(TILES * CHUNK_SLOT) % CHUNK_SLOT,
    CHUNKS_TOTAL)


NBUF = 2


def _sc_agg_body(xa_hbm, xb_hbm, src_hbm, dst_hbm, ha_hbm, hb_hbm,
                 srcp_v, dst_v, sidx0, sidx1, rows0, rows1, sem0, sem1,
                 shared):
    c = lax.axis_index("c")
    s = lax.axis_index("s")
    r0 = s * ROWS_PER_TILE
    start = s * CHUNK_SLOT
    rows = (rows0, rows1)
    sidx = (sidx0, sidx1)
    sems = (sem0, sem1)

    def run(x_hbm, out_hbm):
        pltpu.sync_copy(x_hbm.at[pl.ds(r0, ROWS_PER_TILE)],
                        shared.at[pl.ds(r0, ROWS_PER_TILE)])
        pltpu.sync_copy(src_hbm.at[pl.ds(s * (CHUNK_SLOT // 2),
                                         CHUNK_SLOT // 2)], srcp_v)
        pltpu.sync_copy(dst_hbm.at[pl.ds(start, CHUNK_SLOT)], dst_v)
        plsc.subcore_barrier()

        def gather(g, b):
            base = (g % 2) * (CHUNK // 2)
            for j in range(CHUNK // 32):
                v = srcp_v[g // 2, pl.ds(base + 16 * j, 16)]
                sidx[b][pl.ds(32 * j, 16)] = lax.bitwise_and(v, 0xFFFF)
                sidx[b][pl.ds(32 * j + 16, 16)] = lax.shift_right_logical(v, 16)
            return pltpu.async_copy(x_hbm.at[sidx[b]], rows[b], sems[b])

        def scatter(g, b):
            pltpu.sync_copy(rows[b], shared.at[dst_v.at[g]], add=True)

        descs = {}
        for g in range(NBUF):
            descs[g] = gather(g, g % NBUF)
        for g in range(CHUNKS_BASE):
            b = g % NBUF
            descs.pop(g).wait()
            scatter(g, b)
            if g + NBUF < CHUNKS_BASE:
                descs[g + NBUF] = gather(g + NBUF, b)

        @pl.when(s < CHUNKS_REM)
        def _():
            gather(CHUNKS_BASE, 0).wait()
            scatter(CHUNKS_BASE, 0)

        plsc.subcore_barrier()
        pltpu.sync_copy(shared.at[pl.ds(r0, ROWS_PER_TILE)],
                        out_hbm.at[pl.ds(r0, ROWS_PER_TILE)])

    @pl.when(c == 0)
    def _():
        run(xa_hbm, ha_hbm)

    @pl.when(c == 1)
    def _():
        run(xb_hbm, hb_hbm)


@functools.cache
def _sc_aggregate():
    return pl.kernel(
        _sc_agg_body,
        out_type=[jax.ShapeDtypeStruct((N_PAD, DH), jnp.float32),
                  jax.ShapeDtypeStruct((N_PAD, DH), jnp.float32)],
        mesh=plsc.VectorSubcoreMesh(core_axis_name="c", subcore_axis_name="s"),
        scratch_types=[
            pltpu.VMEM((CHUNK_SLOT // 2, CHUNK), jnp.int32),
            pltpu.VMEM((CHUNK_SLOT, CHUNK), jnp.int32),
            pltpu.VMEM((CHUNK,), jnp.int32),
            pltpu.VMEM((CHUNK,), jnp.int32),
            pltpu.VMEM((CHUNK, DH), jnp.float32),
            pltpu.VMEM((CHUNK, DH), jnp.float32),
            pltpu.SemaphoreType.DMA,
            pltpu.SemaphoreType.DMA,
            pltpu.VMEM_SHARED((N_PAD, DH), jnp.float32),
        ],
    )


def _aggregate(xa, xb, src3, dst3):
    return _sc_aggregate()(xa, xb, src3, dst3)


def kernel(atoms_ids, edge_index, mol_ids, emb, W1, b1, W2, b2, Wjk, bjk,
           Wout, bout):
    src = edge_index[0].astype(jnp.int32)
    dst = edge_index[1].astype(jnp.int32)
    src3 = jnp.concatenate([src, jnp.zeros((CHUNK,), jnp.int32)]
                           ).reshape(CHUNKS_TOTAL + 1, CHUNK)[_ROW_MAP]
    dst3 = jnp.concatenate([dst, jnp.zeros((CHUNK,), jnp.int32)]
                           ).reshape(CHUNKS_TOTAL + 1, CHUNK)[_ROW_MAP]
    sp = src3.reshape(TILES * CHUNK_SLOT, CHUNK // 2, 2)
    src3 = (sp[..., 0] | (sp[..., 1] << 16)
            ).reshape(TILES * CHUNK_SLOT // 2, CHUNK)
    dst3 = (dst3.reshape(TILES * CHUNK_SLOT, CHUNK // 32, 16, 2)
            .transpose(0, 1, 3, 2).reshape(TILES * CHUNK_SLOT, CHUNK))

    ids3 = atoms_ids.astype(jnp.int32).reshape(GRID_N, 1, BN)
    mol3 = mol_ids.astype(jnp.int32).reshape(GRID_N, 1, BN)
    emb_pad = jnp.zeros((128, D), jnp.float32).at[:emb.shape[0]].set(emb)
    wout_pad = jnp.zeros((D, 128), jnp.float32).at[:, :OUT_DIM].set(Wout)
    bout_pad = jnp.zeros((1, 128), jnp.float32).at[0, :OUT_DIM].set(bout)

    xa, xb = _embedding(ids3, emb_pad)
    xs = []
    for l in range(N_LAYERS - 1):
        ha, hb = _aggregate(xa, xb, src3, dst3)
        xa, xb = _layer_mlp(ha, hb, W1[l], b1[l].reshape(1, D),
                            W2[l], b2[l].reshape(1, D))
        xs.append((xa, xb))

    ha, hb = _aggregate(xa, xb, src3, dst3)
    logits_pad = _final(ha, hb, W1[2], b1[2].reshape(1, D),
                        W2[2], b2[2].reshape(1, D),
                        xs[0][0], xs[0][1], xs[1][0], xs[1][1],
                        Wjk, bjk.reshape(1, D), mol3, wout_pad, bout_pad)
    return logits_pad[:, :OUT_DIM]
